# baseline (speedup 1.0000x reference)
import functools

import jax
import jax.numpy as jnp
from jax import lax
from jax.experimental import pallas as pl
from jax.experimental.pallas import tpu as pltpu
from jax.experimental.pallas import tpu_sc as plsc

_BATCH = 16384
_DIM = 64
_LANES = 16
_NCORES = 2
_NSUB = 16
_NWORKERS = _NCORES * _NSUB
_BPW = _BATCH // _NWORKERS
_CHUNK = 64
_NCHUNK = _BPW // _CHUNK
_NGROUP = _CHUNK // _LANES
_NCHB = _DIM // _LANES
_NPAIR = 500000


def _build():
    mesh = plsc.VectorSubcoreMesh(core_axis_name="c", subcore_axis_name="s")

    @functools.partial(
        pl.kernel,
        out_type=jax.ShapeDtypeStruct((_BATCH,), jnp.float32),
        mesh=mesh,
        scratch_types=[
            pltpu.VMEM((_NCHUNK, _CHUNK), jnp.int32),
            pltpu.VMEM((_BPW,), jnp.int32),
            pltpu.VMEM((_BPW,), jnp.int32),
            pltpu.VMEM((2, _CHUNK, 2 * _DIM), jnp.float32),
            pltpu.VMEM((2 * _DIM,), jnp.float32),
            pltpu.VMEM((_LANES * _LANES,), jnp.float32),
            pltpu.VMEM((_BPW,), jnp.float32),
            pltpu.SemaphoreType.DMA,
            pltpu.SemaphoreType.DMA,
        ],
        compiler_params=pltpu.CompilerParams(
            needs_layout_passes=False, use_tc_tiling_on_sc=True),
    )
    def scores_kernel(idx_hbm, sub_hbm, prf_hbm, tbl_hbm, ptab_hbm, out_hbm,
                      idx_v, sub_v, prf_v, rows_v, ptab_v, tmp_v, sc_v,
                      sem0, sem1):
        wid = lax.axis_index("s") * _NCORES + lax.axis_index("c")
        base = wid * _BPW
        sems = [sem0, sem1]

        for c in range(_NCHUNK):
            pltpu.sync_copy(idx_hbm.at[pl.ds(base + c * _CHUNK, _CHUNK)],
                            idx_v.at[c])
        pltpu.sync_copy(sub_hbm.at[pl.ds(base, _BPW)], sub_v)
        pltpu.sync_copy(prf_hbm.at[pl.ds(base, _BPW)], prf_v)
        pltpu.sync_copy(ptab_hbm, ptab_v)

        p0 = [ptab_v[pl.ds(j * _LANES, _LANES)] for j in range(_NCHB)]
        p1 = [ptab_v[pl.ds(_DIM + j * _LANES, _LANES)] for j in range(_NCHB)]
        col = lax.broadcasted_iota(jnp.int32, (_LANES,), 0) * _LANES

        def fire(c, buf):
            pltpu.async_copy(tbl_hbm.at[idx_v.at[c]], rows_v.at[buf],
                             sems[buf])

        def drain(buf):
            pltpu.make_async_copy(tbl_hbm.at[idx_v.at[0]], rows_v.at[buf],
                                  sems[buf]).wait()

        def compute(c, buf):
            def group_body(g, _, buf=buf, c=c):
                rbase = c * _CHUNK + g * _LANES
                pvec = prf_v[pl.ds(rbase, _LANES)]
                svec = sub_v[pl.ds(rbase, _LANES)]
                for r in range(_LANES):
                    li = g * _LANES + r
                    p = pvec[r]
                    off = svec[r]
                    prod = None
                    for j in range(_NCHB):
                        rc = rows_v[buf, li, pl.ds(off + j * _LANES, _LANES)]
                        w = jnp.where(p > 0, p1[j], p0[j])
                        prod = rc * w if prod is None else prod + rc * w
                    plsc.store_scatter(tmp_v, [col + r], prod)
                acc = tmp_v[pl.ds(0, _LANES)]
                for d in range(1, _LANES):
                    acc = acc + tmp_v[pl.ds(d * _LANES, _LANES)]
                sc_v[pl.ds(rbase, _LANES)] = acc
                return 0

            lax.fori_loop(0, _NGROUP, group_body, 0)

        fire(0, 0)
        for c in range(_NCHUNK):
            buf = c % 2
            if c + 1 < _NCHUNK:
                fire(c + 1, 1 - buf)
            drain(buf)
            compute(c, buf)

        pltpu.sync_copy(sc_v, out_hbm.at[pl.ds(base, _BPW)])

    return scores_kernel


_scores_kernel = _build()


def kernel(article_indices, preferences, article_table, preference_table):
    idx = article_indices.astype(jnp.int32)
    prf = preferences.astype(jnp.int32)
    ptab = preference_table.reshape(-1).astype(jnp.float32)
    tbl2 = article_table.reshape(_NPAIR, 2 * _DIM)
    return _scores_kernel(idx // 2, (idx % 2) * _DIM, prf, tbl2, ptab)


# speedup vs baseline: 1.5992x; 1.5992x over previous
import functools

import jax
import jax.numpy as jnp
from jax import lax
from jax.experimental import pallas as pl
from jax.experimental.pallas import tpu as pltpu
from jax.experimental.pallas import tpu_sc as plsc

_BATCH = 16384
_DIM = 64
_LANES = 16
_NCORES = 2
_NSUB = 16
_NWORKERS = _NCORES * _NSUB
_BPW = _BATCH // _NWORKERS
_CHUNK = 32
_NCHUNK = _BPW // _CHUNK
_NGROUP = _CHUNK // _LANES
_NCHB = _DIM // _LANES


def _build():
    mesh = plsc.VectorSubcoreMesh(core_axis_name="c", subcore_axis_name="s")

    @functools.partial(
        pl.kernel,
        out_type=jax.ShapeDtypeStruct((_BATCH,), jnp.float32),
        mesh=mesh,
        scratch_types=[
            pltpu.VMEM((_NCHUNK, _CHUNK), jnp.int32),
            pltpu.VMEM((_BPW,), jnp.int32),
            pltpu.VMEM((_BPW,), jnp.int32),
            pltpu.VMEM((2, _CHUNK, 8, _DIM), jnp.float32),
            pltpu.VMEM((2 * _DIM,), jnp.float32),
            pltpu.VMEM((_LANES * _LANES,), jnp.float32),
            pltpu.VMEM((_BPW,), jnp.float32),
            pltpu.SemaphoreType.DMA,
            pltpu.SemaphoreType.DMA,
        ],
        compiler_params=pltpu.CompilerParams(
            needs_layout_passes=False, use_tc_tiling_on_sc=True),
    )
    def scores_kernel(idx_hbm, sub_hbm, prf_hbm, tbl_hbm, ptab_hbm, out_hbm,
                      idx_v, sub_v, prf_v, rows_v, ptab_v, tmp_v, sc_v,
                      sem0, sem1):
        wid = lax.axis_index("s") * _NCORES + lax.axis_index("c")
        base = wid * _BPW
        sems = [sem0, sem1]

        for c in range(_NCHUNK):
            pltpu.sync_copy(idx_hbm.at[pl.ds(base + c * _CHUNK, _CHUNK)],
                            idx_v.at[c])
        pltpu.sync_copy(sub_hbm.at[pl.ds(base, _BPW)], sub_v)
        pltpu.sync_copy(prf_hbm.at[pl.ds(base, _BPW)], prf_v)
        pltpu.sync_copy(ptab_hbm, ptab_v)

        p0 = [ptab_v[pl.ds(j * _LANES, _LANES)] for j in range(_NCHB)]
        p1 = [ptab_v[pl.ds(_DIM + j * _LANES, _LANES)] for j in range(_NCHB)]
        col = lax.broadcasted_iota(jnp.int32, (_LANES,), 0) * _LANES

        def fire(c, buf):
            for g in range(_NGROUP):
                iv = idx_v[c, pl.ds(g * _LANES, _LANES)]
                for k in range(_LANES):
                    r0 = pl.multiple_of(iv[k] * 8, 8)
                    pltpu.async_copy(tbl_hbm.at[pl.ds(r0, 8)],
                                     rows_v.at[buf, g * _LANES + k],
                                     sems[buf])

        def drain(buf):
            for k in range(_CHUNK):
                pltpu.make_async_copy(tbl_hbm.at[pl.ds(0, 8)],
                                      rows_v.at[buf, k], sems[buf]).wait()

        def compute(c, buf):
            def group_body(g, _, buf=buf, c=c):
                rbase = c * _CHUNK + g * _LANES
                pvec = prf_v[pl.ds(rbase, _LANES)]
                svec = sub_v[pl.ds(rbase, _LANES)]
                for r in range(_LANES):
                    li = g * _LANES + r
                    p = pvec[r]
                    s = svec[r]
                    prod = None
                    for j in range(_NCHB):
                        rc = rows_v[buf, li, s, pl.ds(j * _LANES, _LANES)]
                        w = jnp.where(p > 0, p1[j], p0[j])
                        prod = rc * w if prod is None else prod + rc * w
                    plsc.store_scatter(tmp_v, [col + r], prod)
                acc = tmp_v[pl.ds(0, _LANES)]
                for d in range(1, _LANES):
                    acc = acc + tmp_v[pl.ds(d * _LANES, _LANES)]
                sc_v[pl.ds(rbase, _LANES)] = acc
                return 0

            lax.fori_loop(0, _NGROUP, group_body, 0)

        fire(0, 0)

        def pair_body(cc, _):
            c0 = cc * 2
            fire(c0 + 1, 1)
            drain(0)
            compute(c0, 0)

            @pl.when(cc < _NCHUNK // 2 - 1)
            def _():
                fire(c0 + 2, 0)

            drain(1)
            compute(c0 + 1, 1)
            return 0

        lax.fori_loop(0, _NCHUNK // 2, pair_body, 0)

        pltpu.sync_copy(sc_v, out_hbm.at[pl.ds(base, _BPW)])

    return scores_kernel


_scores_kernel = _build()


def kernel(article_indices, preferences, article_table, preference_table):
    idx = article_indices.astype(jnp.int32)
    prf = preferences.astype(jnp.int32)
    ptab = preference_table.reshape(-1).astype(jnp.float32)
    return _scores_kernel(idx // 8, idx % 8, prf, article_table, ptab)
